# Initial kernel scaffold; baseline (speedup 1.0000x reference)
import jax
import jax.numpy as jnp
from jax.experimental import pallas as pl


def kernel(x, edge_index, Wq, Wk, Wv, Wo):
    raise NotImplementedError("write your pallas kernel here")


# speedup vs baseline: 16.6131x; 16.6131x over previous
import functools

import jax
import jax.numpy as jnp
from jax import lax
from jax.experimental import pallas as pl
from jax.experimental.pallas import tpu as pltpu
from jax.experimental.pallas import tpu_sc as plsc

N = 10000
E = 320000
D = 128
H = 8
HD = 16
SCALE = 1.0 / (HD ** 0.5)
NTILES = 32
EPT = E // NTILES
GB = 80
HB = 40
NB = EPT // GB
NPAD = 10240
RPT = NPAD // 16
ZR = 40
DEN = NPAD * H
DPT = DEN // 16
ZD = 1280
NBLK = E // GB
DPAD = 1024 - GB * H
RBLK = 1000


def _proj_body(x_ref, wq_ref, wk_ref, wv_ref, q_ref, k_ref, v_ref):
    x = x_ref[...]
    dn = (((1,), (1,)), ((), ()))
    q_ref[...] = lax.dot_general(x, wq_ref[...], dn,
                                 preferred_element_type=jnp.float32) * SCALE
    k_ref[...] = lax.dot_general(x, wk_ref[...], dn,
                                 preferred_element_type=jnp.float32)
    v_ref[...] = lax.dot_general(x, wv_ref[...], dn,
                                 preferred_element_type=jnp.float32)


def _proj(x2, wq, wk, wv):
    bs_x = pl.BlockSpec((RBLK, D), lambda i: (i, 0))
    bs_w = pl.BlockSpec((D, D), lambda i: (0, 0))
    sds = jax.ShapeDtypeStruct((N, D), jnp.float32)
    return pl.pallas_call(
        _proj_body,
        grid=(N // RBLK,),
        in_specs=[bs_x, bs_w, bs_w, bs_w],
        out_specs=[bs_x, bs_x, bs_x],
        out_shape=[sds, sds, sds],
    )(x2, wq, wk, wv)


_SC_MESH = plsc.VectorSubcoreMesh(core_axis_name="c", subcore_axis_name="s")

_GATHER_DNUMS = lax.GatherDimensionNumbers(
    offset_dims=(), collapsed_slice_dims=(0,), start_index_map=(0,))


def _lane_gather(x, idx):
    return lax.gather(x, idx[:, None], _GATHER_DNUMS, slice_sizes=(1,),
                      mode=lax.GatherScatterMode.PROMISE_IN_BOUNDS)


@functools.partial(
    pl.kernel,
    mesh=_SC_MESH,
    out_type=(jax.ShapeDtypeStruct((2 * NPAD, D), jnp.float32),
              jax.ShapeDtypeStruct((2 * DEN,), jnp.float32)),
    scratch_types=[
        pltpu.VMEM((2, HB), jnp.int32),
        pltpu.VMEM((2, HB), jnp.int32),
        pltpu.VMEM((HB, D), jnp.float32),
        pltpu.VMEM((HB, D), jnp.float32),
        pltpu.VMEM((HB, D), jnp.float32),
        pltpu.VMEM((HB, D), jnp.float32),
        pltpu.VMEM((8, 128), jnp.int32),
        pltpu.VMEM((8, 128), jnp.float32),
        pltpu.VMEM((ZR, D), jnp.float32),
        pltpu.VMEM((ZD,), jnp.float32),
        pltpu.VMEM_SHARED((NPAD, D), jnp.float32),
        pltpu.VMEM_SHARED((DEN,), jnp.float32),
        pltpu.SemaphoreType.DMA,
        pltpu.SemaphoreType.DMA,
        pltpu.SemaphoreType.DMA,
    ],
)
def _edge_kernel(q_hbm, k_hbm, v_hbm, src_hbm, dst_hbm, didx_hbm,
                 num_hbm, den_hbm,
                 src_v, dst_v, q_v, k_v, v_v, wv_v, didx, dval,
                 z_v, zd_v, acc, accd, sem_q, sem_k, sem_v):
    c = lax.axis_index("c")
    s = lax.axis_index("s")
    zero16 = jnp.zeros((16,), jnp.float32)
    iot = lax.iota(jnp.int32, 16)

    def zrow(r, carry):
        for cc in range(D // 16):
            z_v[r, pl.ds(cc * 16, 16)] = zero16
        return carry
    lax.fori_loop(0, ZR, zrow, 0)

    def zden(r, carry):
        zd_v[pl.ds(r * 16, 16)] = zero16
        return carry
    lax.fori_loop(0, ZD // 16, zden, 0)

    for r in range(GB * H // 128, 8):
        for cc in range(8):
            dval[r, pl.ds(cc * 16, 16)] = zero16

    base_r = s * RPT
    for i in range(RPT // ZR):
        pltpu.sync_copy(z_v, acc.at[pl.ds(base_r + i * ZR, ZR)])
    for i in range(DPT // ZD):
        pltpu.sync_copy(zd_v, accd.at[pl.ds(s * DPT + i * ZD, ZD)])
    plsc.subcore_barrier()

    wid = c * 16 + s
    ebase = wid * EPT

    def group(b, carry):
        off = ebase + b * GB
        pltpu.sync_copy(src_hbm.at[pl.ds(off, HB)], src_v.at[0])
        pltpu.sync_copy(src_hbm.at[pl.ds(off + HB, HB)], src_v.at[1])
        pltpu.sync_copy(dst_hbm.at[pl.ds(off, HB)], dst_v.at[0])
        pltpu.sync_copy(dst_hbm.at[pl.ds(off + HB, HB)], dst_v.at[1])
        pltpu.sync_copy(didx_hbm.at[pl.ds((wid * NB + b) * 8, 8)], didx)

        for half in range(2):
            cq = pltpu.async_copy(q_hbm.at[src_v.at[half]], q_v, sem_q)
            ck = pltpu.async_copy(k_hbm.at[dst_v.at[half]], k_v, sem_k)
            cv = pltpu.async_copy(v_hbm.at[dst_v.at[half]], v_v, sem_v)
            cq.wait()
            ck.wait()
            cv.wait()

            def pair(j, pcarry):
                den = zero16
                for par in range(2):
                    e = 2 * j + par
                    for h in range(H):
                        q = q_v[e, pl.ds(h * HD, HD)]
                        k = k_v[e, pl.ds(h * HD, HD)]
                        p = q * k
                        for sh in (8, 4, 2, 1):
                            p = p + _lane_gather(p, iot ^ sh)
                        ex = jnp.exp(p)
                        v = v_v[e, pl.ds(h * HD, HD)]
                        wv_v[e, pl.ds(h * HD, HD)] = ex * v
                        den = jnp.where(iot == (h + 8 * par), ex, den)
                gp = half * (HB // 2) + j
                dval[gp // 8, pl.ds((gp % 8) * 16, 16)] = den
                return pcarry
            lax.fori_loop(0, HB // 2, pair, 0)

            pltpu.sync_copy(wv_v, acc.at[src_v.at[half]], add=True)

        for t in range(8):
            pltpu.sync_copy(dval.at[t], accd.at[didx.at[t]], add=True)
        return carry
    lax.fori_loop(0, NB, group, 0)

    plsc.subcore_barrier()
    pltpu.sync_copy(acc.at[pl.ds(base_r, RPT)],
                    num_hbm.at[pl.ds(c * NPAD + base_r, RPT)])
    pltpu.sync_copy(accd.at[pl.ds(s * DPT, DPT)],
                    den_hbm.at[pl.ds(c * DEN + s * DPT, DPT)])


def _combine_body(pn_ref, pd_ref, wo_ref, o_ref):
    num = pn_ref[0] + pn_ref[1]
    den8 = pd_ref[0] + pd_ref[1]
    hid = lax.broadcasted_iota(jnp.int32, (H, D), 0)
    col = lax.broadcasted_iota(jnp.int32, (H, D), 1)
    sel = (col // HD == hid).astype(jnp.float32)
    den = lax.dot_general(den8, sel, (((1,), (0,)), ((), ())),
                          preferred_element_type=jnp.float32)
    attn = jnp.where(den > 0, num / den, 0.0)
    o_ref[...] = lax.dot_general(attn, wo_ref[...], (((1,), (1,)), ((), ())),
                                 preferred_element_type=jnp.float32)


def _combine(pn, pd, wo):
    return pl.pallas_call(
        _combine_body,
        grid=(N // RBLK,),
        in_specs=[pl.BlockSpec((2, RBLK, D), lambda i: (0, i, 0)),
                  pl.BlockSpec((2, RBLK, H), lambda i: (0, i, 0)),
                  pl.BlockSpec((D, D), lambda i: (0, 0))],
        out_specs=pl.BlockSpec((RBLK, D), lambda i: (i, 0)),
        out_shape=jax.ShapeDtypeStruct((N, D), jnp.float32),
    )(pn, pd, wo)


def kernel(x, edge_index, Wq, Wk, Wv, Wo):
    x2 = x[0]
    q, k, v = _proj(x2, Wq, Wk, Wv)
    src = edge_index[0].astype(jnp.int32)
    dst = edge_index[1].astype(jnp.int32)
    real = ((src * H)[:, None] + jnp.arange(H, dtype=jnp.int32)).reshape(
        NBLK, GB * H)
    pad = jnp.broadcast_to(
        N * H + jnp.arange(DPAD, dtype=jnp.int32), (NBLK, DPAD))
    didx = jnp.concatenate([real, pad], axis=1).reshape(NBLK * 8, 128)
    pn, pd = _edge_kernel(q, k, v, src, dst, didx)
    pn = pn.reshape(2, NPAD, D)
    pd = pd.reshape(2, NPAD, H)
    out = _combine(pn, pd, Wo)
    return out[None]


# speedup vs baseline: 55.0410x; 3.3131x over previous
import functools

import jax
import jax.numpy as jnp
from jax import lax
from jax.experimental import pallas as pl
from jax.experimental.pallas import tpu as pltpu
from jax.experimental.pallas import tpu_sc as plsc

N = 10000
E = 320000
D = 128
H = 8
HD = 16
SCALE = 1.0 / (HD ** 0.5)
NTILES = 32
EPT = E // NTILES
GB = 200
HB = 40
NH = GB // HB
NB = EPT // GB
NPAD = 10240
RPT = NPAD // 16
ZR = 16
DEN = NPAD * H
DPT = DEN // 16
ZD = 1280
NBLK = E // GB
DROWS = 16
DPAD = DROWS * 128 - GB * H
RBLK = 1000


def _proj_body(x_ref, wq_ref, wk_ref, wv_ref, q_ref, k_ref, v_ref):
    x = x_ref[...]
    dn = (((1,), (1,)), ((), ()))
    q_ref[...] = lax.dot_general(x, wq_ref[...], dn,
                                 preferred_element_type=jnp.float32) * SCALE
    k_ref[...] = lax.dot_general(x, wk_ref[...], dn,
                                 preferred_element_type=jnp.float32)
    v_ref[...] = lax.dot_general(x, wv_ref[...], dn,
                                 preferred_element_type=jnp.float32)


def _proj(x2, wq, wk, wv):
    bs_x = pl.BlockSpec((RBLK, D), lambda i: (i, 0))
    bs_w = pl.BlockSpec((D, D), lambda i: (0, 0))
    sds = jax.ShapeDtypeStruct((N, D), jnp.float32)
    return pl.pallas_call(
        _proj_body,
        grid=(N // RBLK,),
        in_specs=[bs_x, bs_w, bs_w, bs_w],
        out_specs=[bs_x, bs_x, bs_x],
        out_shape=[sds, sds, sds],
    )(x2, wq, wk, wv)


_SC_MESH = plsc.VectorSubcoreMesh(core_axis_name="c", subcore_axis_name="s")

_GATHER_DNUMS = lax.GatherDimensionNumbers(
    offset_dims=(), collapsed_slice_dims=(0,), start_index_map=(0,))

_BITREV = (0, 8, 4, 12, 2, 10, 6, 14, 1, 9, 5, 13, 3, 11, 7, 15)


def _lane_gather(x, idx):
    return lax.gather(x, idx[:, None], _GATHER_DNUMS, slice_sizes=(1,),
                      mode=lax.GatherScatterMode.PROMISE_IN_BOUNDS)


@functools.partial(
    pl.kernel,
    mesh=_SC_MESH,
    out_type=(jax.ShapeDtypeStruct((2 * NPAD, D), jnp.float32),
              jax.ShapeDtypeStruct((2 * DEN,), jnp.float32)),
    scratch_types=[
        pltpu.VMEM((16, HB), jnp.int32),
        pltpu.VMEM((HB, D), jnp.float32),
        pltpu.VMEM((HB, D), jnp.float32),
        pltpu.VMEM((HB, D), jnp.float32),
        pltpu.VMEM((HB, D), jnp.float32),
        pltpu.VMEM((HB, D), jnp.float32),
        pltpu.VMEM((HB, D), jnp.float32),
        pltpu.VMEM((DROWS, 128), jnp.int32),
        pltpu.VMEM((DROWS, 128), jnp.float32),
        pltpu.VMEM((ZR, D), jnp.float32),
        pltpu.VMEM((ZD,), jnp.float32),
        pltpu.VMEM_SHARED((NPAD, D), jnp.float32),
        pltpu.VMEM_SHARED((DEN,), jnp.float32),
        pltpu.SemaphoreType.DMA,
        pltpu.SemaphoreType.DMA,
        pltpu.SemaphoreType.DMA,
        pltpu.SemaphoreType.DMA,
        pltpu.SemaphoreType.DMA,
    ],
)
def _edge_kernel(q_hbm, k_hbm, v_hbm, sd_hbm, didx_hbm,
                 num_hbm, den_hbm,
                 sd_v, q_a, k_a, v_a, q_b, k_b, v_b, didx, dval,
                 z_v, zd_v, acc, accd, sem_a, sem_b, sem_d, sem_i, sem_n):
    c = lax.axis_index("c")
    s = lax.axis_index("s")
    zero16 = jnp.zeros((16,), jnp.float32)
    iot = lax.iota(jnp.int32, 16)
    bufs = ((q_a, k_a, v_a, sem_a), (q_b, k_b, v_b, sem_b))

    def zrow(r, carry):
        for cc in range(D // 16):
            z_v[r, pl.ds(cc * 16, 16)] = zero16
        return carry
    lax.fori_loop(0, ZR, zrow, 0)

    def zden(r, carry):
        zd_v[pl.ds(r * 16, 16)] = zero16
        return carry
    lax.fori_loop(0, ZD // 16, zden, 0)

    for r in range(GB * H // 128, DROWS):
        for cc in range(8):
            dval[r, pl.ds(cc * 16, 16)] = zero16

    base_r = s * RPT
    zdescs = [pltpu.async_copy(z_v, acc.at[pl.ds(base_r + i * ZR, ZR)],
                               sem_i)
              for i in range(RPT // ZR)]
    zdescs += [pltpu.async_copy(zd_v, accd.at[pl.ds(s * DPT + i * ZD, ZD)],
                                sem_d)
               for i in range(DPT // ZD)]
    for d in zdescs:
        d.wait()
    plsc.subcore_barrier()

    wid = c * 16 + s

    def group(b, carry):
        grow = (wid * NB + b) * 16
        ci = (pltpu.async_copy(sd_hbm.at[pl.ds(grow, 16)], sd_v, sem_i),
              pltpu.async_copy(didx_hbm.at[pl.ds(grow, DROWS)], didx, sem_i))
        ci[0].wait()
        ci[1].wait()

        def fire(h):
            qb, kb, vb, sem = bufs[h % 2]
            return (pltpu.async_copy(q_hbm.at[sd_v.at[h]], qb, sem),
                    pltpu.async_copy(k_hbm.at[sd_v.at[8 + h]], kb, sem),
                    pltpu.async_copy(v_hbm.at[sd_v.at[8 + h]], vb, sem))

        descs = fire(0)
        nscat = [None, None]
        for half in range(NH):
            qb, kb, vb, _ = bufs[half % 2]
            for d in descs:
                d.wait()
            if half + 1 < NH:
                np_ = (half + 1) % 2
                if nscat[np_] is not None:
                    nscat[np_].wait()
                    nscat[np_] = None
                descs = fire(half + 1)

            def pair(jj, pcarry):
                for sub in range(2):
                    j = 2 * jj + sub
                    vecs = []
                    for par in range(2):
                        e = 2 * j + par
                        for h in range(H):
                            q = qb[e, pl.ds(h * HD, HD)]
                            k = kb[e, pl.ds(h * HD, HD)]
                            vecs.append(q * k)
                    seg = 16
                    while seg > 1:
                        sh = seg // 2
                        vecs = [v + _lane_gather(v, iot ^ sh) for v in vecs]
                        m = (iot % seg) < sh
                        vecs = [jnp.where(m, vecs[i], vecs[i + 1])
                                for i in range(0, len(vecs), 2)]
                        seg = sh
                    ex = jnp.exp(vecs[0])
                    for par in range(2):
                        e = 2 * j + par
                        for h in range(H):
                            lane = _BITREV[par * 8 + h]
                            bex = _lane_gather(
                                ex, jnp.full((16,), lane, jnp.int32))
                            vb[e, pl.ds(h * HD, HD)] = (
                                bex * vb[e, pl.ds(h * HD, HD)])
                    gp = half * (HB // 2) + j
                    dval[gp // 8, pl.ds((gp % 8) * 16, 16)] = ex
                return pcarry
            lax.fori_loop(0, HB // 4, pair, 0)

            nscat[half % 2] = pltpu.async_copy(
                vb, acc.at[sd_v.at[half]], sem_n, add=True)

        ddescs = [pltpu.async_copy(dval.at[t], accd.at[didx.at[t]], sem_d,
                                   add=True)
                  for t in range(-(-GB * H // 128))]
        for d in nscat:
            if d is not None:
                d.wait()
        for d in ddescs:
            d.wait()
        return carry
    lax.fori_loop(0, NB, group, 0)

    plsc.subcore_barrier()
    pltpu.sync_copy(acc.at[pl.ds(base_r, RPT)],
                    num_hbm.at[pl.ds(c * NPAD + base_r, RPT)])
    pltpu.sync_copy(accd.at[pl.ds(s * DPT, DPT)],
                    den_hbm.at[pl.ds(c * DEN + s * DPT, DPT)])


def _combine_body(pn_ref, pd_ref, wo_ref, o_ref):
    num = pn_ref[0] + pn_ref[1]
    den8 = pd_ref[0] + pd_ref[1]
    hid = lax.broadcasted_iota(jnp.int32, (H, D), 0)
    col = lax.broadcasted_iota(jnp.int32, (H, D), 1)
    sel = (col // HD == hid).astype(jnp.float32)
    den = lax.dot_general(den8, sel, (((1,), (0,)), ((), ())),
                          preferred_element_type=jnp.float32)
    attn = jnp.where(den > 0, num / den, 0.0)
    o_ref[...] = lax.dot_general(attn, wo_ref[...], (((1,), (1,)), ((), ())),
                                 preferred_element_type=jnp.float32)


def _combine(pn, pd, wo):
    return pl.pallas_call(
        _combine_body,
        grid=(N // RBLK,),
        in_specs=[pl.BlockSpec((2, RBLK, D), lambda i: (0, i, 0)),
                  pl.BlockSpec((2, RBLK, H), lambda i: (0, i, 0)),
                  pl.BlockSpec((D, D), lambda i: (0, 0))],
        out_specs=pl.BlockSpec((RBLK, D), lambda i: (i, 0)),
        out_shape=jax.ShapeDtypeStruct((N, D), jnp.float32),
    )(pn, pd, wo)


def kernel(x, edge_index, Wq, Wk, Wv, Wo):
    x2 = x[0]
    q, k, v = _proj(x2, Wq, Wk, Wv)
    src = edge_index[0].astype(jnp.int32)
    dst = edge_index[1].astype(jnp.int32)
    zpad = jnp.zeros((NBLK, 3, HB), jnp.int32)
    srows = jnp.concatenate([src.reshape(NBLK, NH, HB), zpad], axis=1)
    drows = jnp.concatenate([dst.reshape(NBLK, NH, HB), zpad], axis=1)
    sd = jnp.concatenate([srows, drows], axis=1).reshape(NBLK * 16, HB)
    brev = jnp.array(_BITREV, dtype=jnp.int32)
    srcp = src.reshape(NBLK, GB // 2, 2)
    real = (srcp[:, :, brev // H] * H + (brev % H)[None, None, :]).reshape(
        NBLK, GB * H)
    dpad = jnp.broadcast_to(
        N * H + jnp.arange(DPAD, dtype=jnp.int32), (NBLK, DPAD))
    didx = jnp.concatenate([real, dpad], axis=1).reshape(NBLK * DROWS, 128)
    pn, pd = _edge_kernel(q, k, v, sd, didx)
    pn = pn.reshape(2, NPAD, D)
    pd = pd.reshape(2, NPAD, H)
    out = _combine(pn, pd, Wo)
    return out[None]


# speedup vs baseline: 58.8388x; 1.0690x over previous
import functools

import jax
import jax.numpy as jnp
from jax import lax
from jax.experimental import pallas as pl
from jax.experimental.pallas import tpu as pltpu
from jax.experimental.pallas import tpu_sc as plsc

N = 10000
E = 320000
D = 128
H = 8
HD = 16
SCALE = 1.0 / (HD ** 0.5)
NTILES = 32
EPT = E // NTILES
GB = 200
HB = 40
NH = GB // HB
NB = EPT // GB
NPAD = 10240
RPT = NPAD // 16
ZR = 16
DEN = NPAD * H
DPT = DEN // 16
ZD = 1280
NBLK = E // GB
DROWS = 16
DPAD = DROWS * 128 - GB * H
RBLK = 1000


def _proj_body(x_ref, wq_ref, wk_ref, wv_ref, q_ref, k_ref, v_ref):
    x = x_ref[...]
    dn = (((1,), (1,)), ((), ()))
    q_ref[...] = lax.dot_general(x, wq_ref[...], dn,
                                 preferred_element_type=jnp.float32) * SCALE
    k_ref[...] = lax.dot_general(x, wk_ref[...], dn,
                                 preferred_element_type=jnp.float32)
    v_ref[...] = lax.dot_general(x, wv_ref[...], dn,
                                 preferred_element_type=jnp.float32)


def _proj(x2, wq, wk, wv):
    bs_x = pl.BlockSpec((RBLK, D), lambda i: (i, 0))
    bs_w = pl.BlockSpec((D, D), lambda i: (0, 0))
    sds = jax.ShapeDtypeStruct((N, D), jnp.float32)
    return pl.pallas_call(
        _proj_body,
        grid=(N // RBLK,),
        in_specs=[bs_x, bs_w, bs_w, bs_w],
        out_specs=[bs_x, bs_x, bs_x],
        out_shape=[sds, sds, sds],
    )(x2, wq, wk, wv)


_SC_MESH = plsc.VectorSubcoreMesh(core_axis_name="c", subcore_axis_name="s")

_GATHER_DNUMS = lax.GatherDimensionNumbers(
    offset_dims=(), collapsed_slice_dims=(0,), start_index_map=(0,))

_BITREV = (0, 8, 4, 12, 2, 10, 6, 14, 1, 9, 5, 13, 3, 11, 7, 15)


def _lane_gather(x, idx):
    return lax.gather(x, idx[:, None], _GATHER_DNUMS, slice_sizes=(1,),
                      mode=lax.GatherScatterMode.PROMISE_IN_BOUNDS)


@functools.partial(
    pl.kernel,
    mesh=_SC_MESH,
    out_type=(jax.ShapeDtypeStruct((2 * NPAD, D), jnp.float32),
              jax.ShapeDtypeStruct((2 * DEN,), jnp.float32)),
    scratch_types=[
        pltpu.VMEM((16, HB), jnp.int32),
        pltpu.VMEM((HB, D), jnp.float32),
        pltpu.VMEM((HB, D), jnp.float32),
        pltpu.VMEM((HB, D), jnp.float32),
        pltpu.VMEM((HB, D), jnp.float32),
        pltpu.VMEM((HB, D), jnp.float32),
        pltpu.VMEM((HB, D), jnp.float32),
        pltpu.VMEM((DROWS, 128), jnp.int32),
        pltpu.VMEM((DROWS, 128), jnp.float32),
        pltpu.VMEM((ZR, D), jnp.float32),
        pltpu.VMEM((ZD,), jnp.float32),
        pltpu.VMEM_SHARED((NPAD, D), jnp.float32),
        pltpu.VMEM_SHARED((DEN,), jnp.float32),
        pltpu.SemaphoreType.DMA,
        pltpu.SemaphoreType.DMA,
        pltpu.SemaphoreType.DMA,
        pltpu.SemaphoreType.DMA,
        pltpu.SemaphoreType.DMA,
    ],
)
def _edge_kernel(q_hbm, k_hbm, v_hbm, sd_hbm, didx_hbm,
                 num_hbm, den_hbm,
                 sd_v, q_a, k_a, v_a, q_b, k_b, v_b, didx, dval,
                 z_v, zd_v, acc, accd, sem_a, sem_b, sem_d, sem_i, sem_n):
    c = lax.axis_index("c")
    s = lax.axis_index("s")
    zero16 = jnp.zeros((16,), jnp.float32)
    iot = lax.iota(jnp.int32, 16)
    bufs = ((q_a, k_a, v_a, sem_a), (q_b, k_b, v_b, sem_b))

    def zrow(r, carry):
        for cc in range(D // 16):
            z_v[r, pl.ds(cc * 16, 16)] = zero16
        return carry
    lax.fori_loop(0, ZR, zrow, 0)

    def zden(r, carry):
        zd_v[pl.ds(r * 16, 16)] = zero16
        return carry
    lax.fori_loop(0, ZD // 16, zden, 0)

    for r in range(GB * H // 128, DROWS):
        for cc in range(8):
            dval[r, pl.ds(cc * 16, 16)] = zero16

    base_r = s * RPT
    zdescs = [pltpu.async_copy(z_v, acc.at[pl.ds(base_r + i * ZR, ZR)],
                               sem_i)
              for i in range(RPT // ZR)]
    zdescs += [pltpu.async_copy(zd_v, accd.at[pl.ds(s * DPT + i * ZD, ZD)],
                                sem_d)
               for i in range(DPT // ZD)]
    for d in zdescs:
        d.wait()
    plsc.subcore_barrier()

    wid = c * 16 + s

    def group(b, carry):
        grow = (wid * NB + b) * 16
        ci = (pltpu.async_copy(sd_hbm.at[pl.ds(grow, 16)], sd_v, sem_i),
              pltpu.async_copy(didx_hbm.at[pl.ds(grow, DROWS)], didx, sem_i))
        ci[0].wait()
        ci[1].wait()

        def fire(h):
            qb, kb, vb, sem = bufs[h % 2]
            return (pltpu.async_copy(q_hbm.at[sd_v.at[h]], qb, sem),
                    pltpu.async_copy(k_hbm.at[sd_v.at[8 + h]], kb, sem),
                    pltpu.async_copy(v_hbm.at[sd_v.at[8 + h]], vb, sem))

        descs = fire(0)
        nscat = [None, None]
        for half in range(NH):
            qb, kb, vb, _ = bufs[half % 2]
            for d in descs:
                d.wait()
            if half + 1 < NH:
                np_ = (half + 1) % 2
                if nscat[np_] is not None:
                    nscat[np_].wait()
                    nscat[np_] = None
                descs = fire(half + 1)

            def pair(j, pcarry):
                vecs = []
                for par in range(2):
                    e = 2 * j + par
                    for h in range(H):
                        q = qb[e, pl.ds(h * HD, HD)]
                        k = kb[e, pl.ds(h * HD, HD)]
                        vecs.append(q * k)
                seg = 16
                while seg > 1:
                    sh = seg // 2
                    vecs = [v + _lane_gather(v, iot ^ sh) for v in vecs]
                    m = (iot % seg) < sh
                    vecs = [jnp.where(m, vecs[i], vecs[i + 1])
                            for i in range(0, len(vecs), 2)]
                    seg = sh
                ex = jnp.exp(vecs[0])
                for par in range(2):
                    e = 2 * j + par
                    for h in range(H):
                        lane = _BITREV[par * 8 + h]
                        bex = _lane_gather(
                            ex, jnp.full((16,), lane, jnp.int32))
                        vb[e, pl.ds(h * HD, HD)] = (
                            bex * vb[e, pl.ds(h * HD, HD)])
                gp = half * (HB // 2) + j
                dval[gp // 8, pl.ds((gp % 8) * 16, 16)] = ex
                return pcarry
            lax.fori_loop(0, HB // 2, pair, 0)

            nscat[half % 2] = pltpu.async_copy(
                vb, acc.at[sd_v.at[half]], sem_n, add=True)

        ddescs = [pltpu.async_copy(dval.at[t], accd.at[didx.at[t]], sem_d,
                                   add=True)
                  for t in range(DROWS)]
        for d in nscat:
            if d is not None:
                d.wait()
        for d in ddescs:
            d.wait()
        return carry
    lax.fori_loop(0, NB, group, 0)

    plsc.subcore_barrier()
    pltpu.sync_copy(acc.at[pl.ds(base_r, RPT)],
                    num_hbm.at[pl.ds(c * NPAD + base_r, RPT)])
    pltpu.sync_copy(accd.at[pl.ds(s * DPT, DPT)],
                    den_hbm.at[pl.ds(c * DEN + s * DPT, DPT)])


def _combine_body(pn_ref, pd_ref, wo_ref, o_ref):
    num = pn_ref[0] + pn_ref[1]
    den8 = pd_ref[0] + pd_ref[1]
    hid = lax.broadcasted_iota(jnp.int32, (H, D), 0)
    col = lax.broadcasted_iota(jnp.int32, (H, D), 1)
    sel = (col // HD == hid).astype(jnp.float32)
    den = lax.dot_general(den8, sel, (((1,), (0,)), ((), ())),
                          preferred_element_type=jnp.float32)
    attn = jnp.where(den > 0, num / den, 0.0)
    o_ref[...] = lax.dot_general(attn, wo_ref[...], (((1,), (1,)), ((), ())),
                                 preferred_element_type=jnp.float32)


def _combine(pn, pd, wo):
    return pl.pallas_call(
        _combine_body,
        grid=(N // RBLK,),
        in_specs=[pl.BlockSpec((2, RBLK, D), lambda i: (0, i, 0)),
                  pl.BlockSpec((2, RBLK, H), lambda i: (0, i, 0)),
                  pl.BlockSpec((D, D), lambda i: (0, 0))],
        out_specs=pl.BlockSpec((RBLK, D), lambda i: (i, 0)),
        out_shape=jax.ShapeDtypeStruct((N, D), jnp.float32),
    )(pn, pd, wo)


def kernel(x, edge_index, Wq, Wk, Wv, Wo):
    x2 = x[0]
    q, k, v = _proj(x2, Wq, Wk, Wv)
    src = edge_index[0].astype(jnp.int32)
    dst = edge_index[1].astype(jnp.int32)
    zpad = jnp.zeros((NBLK, 3, HB), jnp.int32)
    srows = jnp.concatenate([src.reshape(NBLK, NH, HB), zpad], axis=1)
    drows = jnp.concatenate([dst.reshape(NBLK, NH, HB), zpad], axis=1)
    sd = jnp.concatenate([srows, drows], axis=1).reshape(NBLK * 16, HB)
    brev = jnp.array(_BITREV, dtype=jnp.int32)
    srcp = src.reshape(NBLK, GB // 2, 2)
    real = (srcp[:, :, brev // H] * H + (brev % H)[None, None, :]).reshape(
        NBLK, GB * H)
    dpad = jnp.broadcast_to(
        N * H + jnp.arange(DPAD, dtype=jnp.int32), (NBLK, DPAD))
    didx = jnp.concatenate([real, dpad], axis=1).reshape(NBLK * DROWS, 128)
    pn, pd = _edge_kernel(q, k, v, sd, didx)
    pn = pn.reshape(2, NPAD, D)
    pd = pd.reshape(2, NPAD, H)
    out = _combine(pn, pd, Wo)
    return out[None]


# speedup vs baseline: 59.2584x; 1.0071x over previous
import functools

import jax
import jax.numpy as jnp
from jax import lax
from jax.experimental import pallas as pl
from jax.experimental.pallas import tpu as pltpu
from jax.experimental.pallas import tpu_sc as plsc

N = 10000
E = 320000
D = 128
H = 8
HD = 16
SCALE = 1.0 / (HD ** 0.5)
NTILES = 32
EPT = E // NTILES
GB = 200
HB = 40
NH = GB // HB
NB = EPT // GB
NPAD = 10240
RPT = NPAD // 16
ZR = 16
DEN = NPAD * H
DPT = DEN // 16
ZD = 1280
NBLK = E // GB
DROWS = 16
DPAD = DROWS * 128 - GB * H
RBLK = 1000


def _proj_body(x_ref, wq_ref, wk_ref, wv_ref, q_ref, k_ref, v_ref):
    x = x_ref[...]
    dn = (((1,), (1,)), ((), ()))
    q_ref[...] = lax.dot_general(x, wq_ref[...], dn,
                                 preferred_element_type=jnp.float32) * SCALE
    k_ref[...] = lax.dot_general(x, wk_ref[...], dn,
                                 preferred_element_type=jnp.float32)
    v_ref[...] = lax.dot_general(x, wv_ref[...], dn,
                                 preferred_element_type=jnp.float32)


def _proj(x2, wq, wk, wv):
    bs_x = pl.BlockSpec((RBLK, D), lambda i: (i, 0))
    bs_w = pl.BlockSpec((D, D), lambda i: (0, 0))
    sds = jax.ShapeDtypeStruct((N, D), jnp.float32)
    return pl.pallas_call(
        _proj_body,
        grid=(N // RBLK,),
        in_specs=[bs_x, bs_w, bs_w, bs_w],
        out_specs=[bs_x, bs_x, bs_x],
        out_shape=[sds, sds, sds],
    )(x2, wq, wk, wv)


_SC_MESH = plsc.VectorSubcoreMesh(core_axis_name="c", subcore_axis_name="s")

_GATHER_DNUMS = lax.GatherDimensionNumbers(
    offset_dims=(), collapsed_slice_dims=(0,), start_index_map=(0,))

_BITREV = (0, 8, 4, 12, 2, 10, 6, 14, 1, 9, 5, 13, 3, 11, 7, 15)


def _lane_gather(x, idx):
    return lax.gather(x, idx[:, None], _GATHER_DNUMS, slice_sizes=(1,),
                      mode=lax.GatherScatterMode.PROMISE_IN_BOUNDS)


@functools.partial(
    pl.kernel,
    mesh=_SC_MESH,
    out_type=(jax.ShapeDtypeStruct((2 * NPAD, D), jnp.float32),
              jax.ShapeDtypeStruct((2 * DEN,), jnp.float32)),
    scratch_types=[
        pltpu.VMEM((16, HB), jnp.int32),
        pltpu.VMEM((HB, D), jnp.float32),
        pltpu.VMEM((HB, D), jnp.float32),
        pltpu.VMEM((HB, D), jnp.float32),
        pltpu.VMEM((HB, D), jnp.float32),
        pltpu.VMEM((HB, D), jnp.float32),
        pltpu.VMEM((HB, D), jnp.float32),
        pltpu.VMEM((DROWS, 128), jnp.int32),
        pltpu.VMEM((DROWS, 128), jnp.float32),
        pltpu.VMEM((ZR, D), jnp.float32),
        pltpu.VMEM((ZD,), jnp.float32),
        pltpu.VMEM_SHARED((NPAD, D), jnp.float32),
        pltpu.VMEM_SHARED((DEN,), jnp.float32),
        pltpu.SemaphoreType.DMA,
        pltpu.SemaphoreType.DMA,
        pltpu.SemaphoreType.DMA,
        pltpu.SemaphoreType.DMA,
        pltpu.SemaphoreType.DMA,
    ],
)
def _edge_kernel(q_hbm, k_hbm, v_hbm, sd_hbm, didx_hbm,
                 num_hbm, den_hbm,
                 sd_v, q_a, k_a, v_a, q_b, k_b, v_b, didx, dval,
                 z_v, zd_v, acc, accd, sem_a, sem_b, sem_d, sem_i, sem_n):
    c = lax.axis_index("c")
    s = lax.axis_index("s")
    zero16 = jnp.zeros((16,), jnp.float32)
    iot = lax.iota(jnp.int32, 16)
    bufs = ((q_a, k_a, v_a, sem_a), (q_b, k_b, v_b, sem_b))

    def zrow(r, carry):
        for cc in range(D // 16):
            z_v[r, pl.ds(cc * 16, 16)] = zero16
        return carry
    lax.fori_loop(0, ZR, zrow, 0)

    def zden(r, carry):
        zd_v[pl.ds(r * 16, 16)] = zero16
        return carry
    lax.fori_loop(0, ZD // 16, zden, 0)

    for r in range(GB * H // 128, DROWS):
        for cc in range(8):
            dval[r, pl.ds(cc * 16, 16)] = zero16

    base_r = s * RPT
    zdescs = [pltpu.async_copy(z_v, acc.at[pl.ds(base_r + i * ZR, ZR)],
                               sem_i)
              for i in range(RPT // ZR)]
    zdescs += [pltpu.async_copy(zd_v, accd.at[pl.ds(s * DPT + i * ZD, ZD)],
                                sem_d)
               for i in range(DPT // ZD)]
    for d in zdescs:
        d.wait()
    plsc.subcore_barrier()

    wid = c * 16 + s

    def group(b, carry):
        grow = (wid * NB + b) * 16
        ci = (pltpu.async_copy(sd_hbm.at[pl.ds(grow, 16)], sd_v, sem_i),
              pltpu.async_copy(didx_hbm.at[pl.ds(grow, DROWS)], didx, sem_i))
        ci[0].wait()
        ci[1].wait()

        def fire(h):
            qb, kb, vb, sem = bufs[h % 2]
            return (pltpu.async_copy(q_hbm.at[sd_v.at[h]], qb, sem),
                    pltpu.async_copy(k_hbm.at[sd_v.at[8 + h]], kb, sem),
                    pltpu.async_copy(v_hbm.at[sd_v.at[8 + h]], vb, sem))

        descs = fire(0)
        nscat = [None, None]
        for half in range(NH):
            qb, kb, vb, _ = bufs[half % 2]
            for d in descs:
                d.wait()
            if half + 1 < NH:
                np_ = (half + 1) % 2
                if nscat[np_] is not None:
                    nscat[np_].wait()
                    nscat[np_] = None
                descs = fire(half + 1)

            def pair(j, pcarry):
                vecs = []
                for par in range(2):
                    e = 2 * j + par
                    for h in range(H):
                        q = qb[e, pl.ds(h * HD, HD)]
                        k = kb[e, pl.ds(h * HD, HD)]
                        vecs.append(q * k)
                seg = 16
                while seg > 1:
                    sh = seg // 2
                    vecs = [v + _lane_gather(v, iot ^ sh) for v in vecs]
                    m = (iot % seg) < sh
                    vecs = [jnp.where(m, vecs[i], vecs[i + 1])
                            for i in range(0, len(vecs), 2)]
                    seg = sh
                ex = jnp.exp(vecs[0])
                for par in range(2):
                    e = 2 * j + par
                    for h in range(H):
                        lane = _BITREV[par * 8 + h]
                        bex = _lane_gather(
                            ex, jnp.full((16,), lane, jnp.int32))
                        vb[e, pl.ds(h * HD, HD)] = (
                            bex * vb[e, pl.ds(h * HD, HD)])
                gp = half * (HB // 2) + j
                dval[gp // 8, pl.ds((gp % 8) * 16, 16)] = ex
                return pcarry
            lax.fori_loop(0, HB // 2, pair, 0)

            nscat[half % 2] = pltpu.async_copy(
                vb, acc.at[sd_v.at[half]], sem_n, add=True)

        ddescs = [pltpu.async_copy(dval.at[t], accd.at[didx.at[t]], sem_d,
                                   add=True)
                  for t in range(-(-GB * H // 128))]
        for d in nscat:
            if d is not None:
                d.wait()
        for d in ddescs:
            d.wait()
        return carry
    lax.fori_loop(0, NB, group, 0)

    plsc.subcore_barrier()
    pltpu.sync_copy(acc.at[pl.ds(base_r, RPT)],
                    num_hbm.at[pl.ds(c * NPAD + base_r, RPT)])
    pltpu.sync_copy(accd.at[pl.ds(s * DPT, DPT)],
                    den_hbm.at[pl.ds(c * DEN + s * DPT, DPT)])


def _combine_body(pn_ref, pd_ref, wo_ref, o_ref):
    num = pn_ref[0] + pn_ref[1]
    den8 = pd_ref[0] + pd_ref[1]
    hid = lax.broadcasted_iota(jnp.int32, (H, D), 0)
    col = lax.broadcasted_iota(jnp.int32, (H, D), 1)
    sel = (col // HD == hid).astype(jnp.float32)
    den = lax.dot_general(den8, sel, (((1,), (0,)), ((), ())),
                          preferred_element_type=jnp.float32)
    attn = jnp.where(den > 0, num / den, 0.0)
    o_ref[...] = lax.dot_general(attn, wo_ref[...], (((1,), (1,)), ((), ())),
                                 preferred_element_type=jnp.float32)


def _combine(pn, pd, wo):
    return pl.pallas_call(
        _combine_body,
        grid=(N // RBLK,),
        in_specs=[pl.BlockSpec((2, RBLK, D), lambda i: (0, i, 0)),
                  pl.BlockSpec((2, RBLK, H), lambda i: (0, i, 0)),
                  pl.BlockSpec((D, D), lambda i: (0, 0))],
        out_specs=pl.BlockSpec((RBLK, D), lambda i: (i, 0)),
        out_shape=jax.ShapeDtypeStruct((N, D), jnp.float32),
    )(pn, pd, wo)


def kernel(x, edge_index, Wq, Wk, Wv, Wo):
    x2 = x[0]
    q, k, v = _proj(x2, Wq, Wk, Wv)
    src = edge_index[0].astype(jnp.int32)
    dst = edge_index[1].astype(jnp.int32)
    zpad = jnp.zeros((NBLK, 3, HB), jnp.int32)
    srows = jnp.concatenate([src.reshape(NBLK, NH, HB), zpad], axis=1)
    drows = jnp.concatenate([dst.reshape(NBLK, NH, HB), zpad], axis=1)
    sd = jnp.concatenate([srows, drows], axis=1).reshape(NBLK * 16, HB)
    brev = jnp.array(_BITREV, dtype=jnp.int32)
    srcp = src.reshape(NBLK, GB // 2, 2)
    real = (srcp[:, :, brev // H] * H + (brev % H)[None, None, :]).reshape(
        NBLK, GB * H)
    dpad = jnp.broadcast_to(
        N * H + jnp.arange(DPAD, dtype=jnp.int32), (NBLK, DPAD))
    didx = jnp.concatenate([real, dpad], axis=1).reshape(NBLK * DROWS, 128)
    pn, pd = _edge_kernel(q, k, v, sd, didx)
    pn = pn.reshape(2, NPAD, D)
    pd = pd.reshape(2, NPAD, H)
    out = _combine(pn, pd, Wo)
    return out[None]
